# baseline (speedup 1.0000x reference)
import functools

import jax
import jax.numpy as jnp
from jax import lax
from jax.experimental import pallas as pl
from jax.experimental.pallas import tpu as pltpu
from jax.experimental.pallas import tpu_sc as plsc

_FEATURES = 2048
_K = 8
_XMIN = -3.0
_XMAX = 3.0
_DELTA = (_XMAX - _XMIN) / float(_K - 1)
_INV_DELTA = 1.0 / _DELTA
_POS_OFF = -_XMIN * _INV_DELTA
_POS_MAX = float(_K - 1)

_NC = 2
_NS = 16
_NW = _NC * _NS
_LANES = 16

_TAB_PAD = _FEATURES * _K + _LANES


def _sc_body(n_chunks, ch_elems, x_hbm, tab_hbm, out_hbm, tab_v, buf):
    wid = lax.axis_index("s") * _NC + lax.axis_index("c")
    base = wid * (n_chunks * ch_elems)
    pltpu.sync_copy(tab_hbm, tab_v)
    iota8 = lax.iota(jnp.int32, _LANES) * _K
    n_vec = ch_elems // _LANES
    vec_per_row = _FEATURES // _LANES

    def chunk_body(c, carry):
        off = base + c * ch_elems
        pltpu.sync_copy(x_hbm.at[pl.ds(off, ch_elems)], buf)

        @plsc.parallel_loop(0, n_vec, 1, unroll=8)
        def _(i):
            o16 = pl.multiple_of(i * _LANES, _LANES)
            v = buf[pl.ds(o16, _LANES)]
            pos = jnp.minimum(
                jnp.maximum(v * _INV_DELTA + _POS_OFF, 0.0), _POS_MAX)
            i0 = pos.astype(jnp.int32)
            frac = pos - i0.astype(jnp.float32)
            fbase = lax.rem(i, vec_per_row) * (_LANES * _K)
            idx0 = iota8 + fbase + i0
            y0 = plsc.load_gather(tab_v, [idx0])
            y1 = plsc.load_gather(tab_v, [idx0 + 1])
            buf[pl.ds(o16, _LANES)] = y0 + frac * (y1 - y0)

        pltpu.sync_copy(buf, out_hbm.at[pl.ds(off, ch_elems)])
        return carry

    lax.fori_loop(0, n_chunks, chunk_body, 0)


def _sc_spline(x_flat, tab, ch_rows=32):
    total = x_flat.size
    per_worker = total // _NW
    ch_elems = ch_rows * _FEATURES
    n_chunks = per_worker // ch_elems
    mesh = plsc.VectorSubcoreMesh(
        core_axis_name="c", subcore_axis_name="s",
        num_cores=_NC, num_subcores=_NS)
    fn = pl.kernel(
        functools.partial(_sc_body, n_chunks, ch_elems),
        out_type=jax.ShapeDtypeStruct((total,), jnp.float32),
        mesh=mesh,
        scratch_types=[
            pltpu.VMEM((_TAB_PAD,), jnp.float32),
            pltpu.VMEM((ch_elems,), jnp.float32),
        ],
        compiler_params=pltpu.CompilerParams(needs_layout_passes=False),
    )
    return fn(x_flat, tab)


def _tc_spline_body(x_ref, ky_ref, o_ref):
    x = x_ref[...]
    xc = jnp.clip(x, _XMIN, _XMAX)
    pos = (xc - _XMIN) * _INV_DELTA
    idx0 = jnp.minimum(pos.astype(jnp.int32), _K - 2)
    frac = pos - idx0.astype(jnp.float32)
    y0 = ky_ref[0:1, :]
    y1 = ky_ref[1:2, :]
    for k in range(1, _K - 1):
        m = idx0 >= k
        y0 = jnp.where(m, ky_ref[k:k + 1, :], y0)
        y1 = jnp.where(m, ky_ref[k + 1:k + 2, :], y1)
    o_ref[...] = y0 + frac * (y1 - y0)


def _tc_spline(flat, kyT, br=512):
    rows = flat.shape[0]
    return pl.pallas_call(
        _tc_spline_body,
        grid=(rows // br,),
        in_specs=[
            pl.BlockSpec((br, _FEATURES), lambda i: (i, 0)),
            pl.BlockSpec((_K, _FEATURES), lambda i: (0, 0)),
        ],
        out_specs=pl.BlockSpec((br, _FEATURES), lambda i: (i, 0)),
        out_shape=jax.ShapeDtypeStruct(flat.shape, flat.dtype),
    )(flat, kyT)


def kernel(x, knot_y):
    tab = jnp.concatenate(
        [knot_y.reshape(-1), jnp.zeros((_LANES,), jnp.float32)])
    out_flat = _sc_spline(x.reshape(-1), tab)
    return out_flat.reshape(x.shape)
